# baseline (speedup 1.0000x reference)
import jax
import jax.numpy as jnp
from jax.experimental import pallas as pl
from jax.experimental.pallas import tpu as pltpu

_P = 512
_R = 0.05
_TI = 32
_F = 32
_C = 4
_L = _F * _C
_Q = _P // _C
_NSUB = 8
_QT = _Q // _NSUB


def _grad_body(xr_ref, permS_ref, permL_ref, Wa4_ref, Wb16_ref, sjx_ref,
               sjy_ref, b1_4_ref, w1d4_ref, W2b_ref, W2bT_ref, b2_4_ref,
               wo4_ref, VBS_ref, V1X_ref, V1Y_ref, V2X_ref, V2Y_ref,
               RED4_ref, PK_ref, SP_ref, UR_ref, EC_ref,
               g_ref, Pm_s, PmT_s, xs_s, xs4_s, gsx_s, gsy_s, gsrc_s):
    it = pl.program_id(1)
    nit = pl.num_programs(1)
    xb = xr_ref[0]

    @pl.when(it == 0)
    def _():
        permS = permS_ref[0]
        permL = permL_ref[0]
        col = jax.lax.broadcasted_iota(jnp.int32, (_P, _P), 1)
        row = jax.lax.broadcasted_iota(jnp.int32, (_P, _P), 0)
        Pm_s[...] = jnp.where(permS == col, 1.0, 0.0)
        PmT_s[...] = jnp.where(permL == row, 1.0, 0.0)
        xs = jnp.dot(Pm_s[...], xb, preferred_element_type=jnp.float32)
        xs_s[...] = xs
        xs4 = jnp.zeros((_Q, 16), jnp.float32)
        for c in range(_C):
            xc = jnp.dot(PK_ref[c], xs, preferred_element_type=jnp.float32)
            xs4 = xs4 + jnp.dot(xc, SP_ref[c],
                                preferred_element_type=jnp.float32)
        xs4_s[...] = xs4
        gsx_s[...] = jnp.zeros_like(gsx_s)
        gsy_s[...] = jnp.zeros_like(gsy_s)
        gsrc_s[...] = jnp.zeros_like(gsrc_s)

    xi = xs_s[pl.ds(it * _TI, _TI), :]
    A4 = jnp.dot(xi, Wa4_ref[...], preferred_element_type=jnp.float32) + b1_4_ref[...]
    pix = jnp.broadcast_to(xi[:, 0:1], (_TI, _L))
    piy = jnp.broadcast_to(xi[:, 1:2], (_TI, _L))
    i_lo_x = jnp.min(xi[:, 0])
    i_hi_x = jnp.max(xi[:, 0])
    i_lo_y = jnp.min(xi[:, 1])
    i_hi_y = jnp.max(xi[:, 1])
    lane16 = jax.lax.broadcasted_iota(jnp.int32, (_QT, 16), 1) % 4
    w1d4 = w1d4_ref[...]
    W2b = W2b_ref[...]
    W2bT = W2bT_ref[...]
    RED4 = RED4_ref[...]

    def do_subtile(jt):
        xj4 = xs4_s[jt * _QT:(jt + 1) * _QT, :]
        inf = jnp.float32(jnp.inf)
        j_lo_x = jnp.min(jnp.where(lane16 == 0, xj4, inf))
        j_hi_x = jnp.max(jnp.where(lane16 == 0, xj4, -inf))
        j_lo_y = jnp.min(jnp.where(lane16 == 1, xj4, inf))
        j_hi_y = jnp.max(jnp.where(lane16 == 1, xj4, -inf))
        gx = jnp.maximum(jnp.maximum(i_lo_x - j_hi_x, j_lo_x - i_hi_x), 0.0)
        gy = jnp.maximum(jnp.maximum(i_lo_y - j_hi_y, j_lo_y - i_hi_y), 0.0)

        @pl.when(gx * gx + gy * gy < _R * _R)
        def _():
            B4 = jnp.dot(xj4, Wb16_ref[...], preferred_element_type=jnp.float32)
            pjx = jnp.dot(xj4, sjx_ref[...], preferred_element_type=jnp.float32)
            pjy = jnp.dot(xj4, sjy_ref[...], preferred_element_type=jnp.float32)

            relx = pix[:, None, :] - pjx[None, :, :]
            rely = piy[:, None, :] - pjy[None, :, :]
            dist2 = ((pix * pix + piy * piy)[:, None, :]
                     + (pjx * pjx + pjy * pjy)[None, :, :]
                     - 2.0 * (pix[:, None, :] * pjx[None, :, :]
                              + piy[:, None, :] * pjy[None, :, :]))
            j_id = (4 * (jt * _QT
                         + jax.lax.broadcasted_iota(jnp.int32, (_QT, _L), 0))
                    + jax.lax.broadcasted_iota(jnp.int32, (_QT, _L), 1) // _F)
            i_id = it * _TI + jax.lax.broadcasted_iota(
                jnp.int32, (_TI, _QT, _L), 0)
            mask = (dist2 < _R * _R) & (i_id != j_id[None, :, :])
            r2 = relx * relx + rely * rely + 1e-8
            rdist = jax.lax.rsqrt(r2)
            dist = r2 * rdist

            z1 = (A4[:, None, :] + B4[None, :, :]
                  + dist * w1d4[0][None, None, :])
            h = jnp.tanh(z1).reshape(_TI * _QT, _L)
            z2 = jnp.dot(h, W2b, preferred_element_type=jnp.float32) + b2_4_ref[...]
            t2 = jnp.tanh(z2)
            maskf = mask.reshape(_TI * _QT, _L)
            dz2 = jnp.where(maskf, (1.0 - t2 * t2) * wo4_ref[...], 0.0)
            dh = jnp.dot(dz2, W2bT, preferred_element_type=jnp.float32)
            dz1 = dh * (1.0 - h * h)

            def red(v_ref):
                r = jnp.dot(dz1, v_ref[...], preferred_element_type=jnp.float32)
                return r.reshape(_TI, _QT, _L)

            srd = red(VBS_ref) * rdist
            sux = srd * relx
            suy = srd * rely
            v1x = red(V1X_ref) + sux
            v1y = red(V1Y_ref) + suy
            v2x = red(V2X_ref) - sux
            v2y = red(V2Y_ref) - suy

            gi_x = jnp.sum(jnp.sum(v1x, axis=2), axis=1, keepdims=True) * (1.0 / _F)
            gi_y = jnp.sum(jnp.sum(v1y, axis=2), axis=1, keepdims=True) * (1.0 / _F)
            gsx_s[pl.ds(it * _TI, _TI), :] += gi_x
            gsy_s[pl.ds(it * _TI, _TI), :] += gi_y
            gj4 = jnp.concatenate(
                [jnp.dot(jnp.sum(v2x, axis=0), RED4,
                         preferred_element_type=jnp.float32),
                 jnp.dot(jnp.sum(v2y, axis=0), RED4,
                         preferred_element_type=jnp.float32)], axis=1)
            gsrc_s[jt * _QT:(jt + 1) * _QT, :] += gj4

    for jt in range(_NSUB):
        do_subtile(jt)

    @pl.when(it == nit - 1)
    def _():
        gsort = jnp.concatenate([gsx_s[...], gsy_s[...]], axis=1)
        gsrc = gsrc_s[...]
        for c in range(_C):
            part = jnp.dot(gsrc, EC_ref[c], preferred_element_type=jnp.float32)
            gsort = gsort + jnp.dot(UR_ref[c], part,
                                    preferred_element_type=jnp.float32)
        g_ref[0] = jnp.dot(PmT_s[...], gsort,
                           preferred_element_type=jnp.float32)


def _grad_step(xr, permS, permL, consts):
    nb = xr.shape[0]
    grid = (nb, _P // _TI)

    def wspec(a):
        return pl.BlockSpec(a.shape, lambda b, it: (0,) * a.ndim)

    g = pl.pallas_call(
        _grad_body,
        grid=grid,
        in_specs=[
            pl.BlockSpec((1, _P, 4), lambda b, it: (b, 0, 0)),
            pl.BlockSpec((1, _P, 1), lambda b, it: (b, 0, 0)),
            pl.BlockSpec((1, 1, _P), lambda b, it: (b, 0, 0)),
        ] + [wspec(c) for c in consts],
        out_specs=pl.BlockSpec((1, _P, 2), lambda b, it: (b, 0, 0)),
        out_shape=jax.ShapeDtypeStruct((nb, _P, 2), jnp.float32),
        scratch_shapes=[
            pltpu.VMEM((_P, _P), jnp.float32),
            pltpu.VMEM((_P, _P), jnp.float32),
            pltpu.VMEM((_P, 4), jnp.float32),
            pltpu.VMEM((_Q, 16), jnp.float32),
            pltpu.VMEM((_P, 1), jnp.float32),
            pltpu.VMEM((_P, 1), jnp.float32),
            pltpu.VMEM((_Q, 8), jnp.float32),
        ],
        compiler_params=pltpu.CompilerParams(
            dimension_semantics=("parallel", "arbitrary")),
    )(xr, permS, permL, *consts)
    return g


def _morton_codes(pos):
    q = jnp.clip((pos + 0.5) * 1024.0, 0.0, 1023.0).astype(jnp.int32)

    def part(v):
        v = (v | (v << 8)) & 0x00FF00FF
        v = (v | (v << 4)) & 0x0F0F0F0F
        v = (v | (v << 2)) & 0x33333333
        v = (v | (v << 1)) & 0x55555555
        return v

    return part(q[:, 0]) | (part(q[:, 1]) << 1)


def kernel(x, batch, steps, W1, b1, W2, b2, Wout, bout):
    N = x.shape[0]
    nb = N // _P
    f32 = jnp.float32

    Wr = W1[8:10]
    pad = jnp.zeros((2, _F), dtype=f32)
    Wa = W1[0:4] + jnp.concatenate([Wr, pad], axis=0)
    Wb = W1[4:8] - jnp.concatenate([Wr, pad], axis=0)
    w1d = W1[10:11]
    c1x = W1[0] + W1[8]
    c1y = W1[1] + W1[9]
    c2x = W1[4] - W1[8]
    c2y = W1[5] - W1[9]

    eye4 = jnp.eye(_C, dtype=f32)
    eyeQ = jnp.eye(_Q, dtype=f32)
    ones1F = jnp.ones((1, _F), dtype=f32)

    def bcmat(vec):
        return jnp.kron(eye4, vec[:, None] @ ones1F)

    def e(c, n):
        return jnp.zeros((n, 1), dtype=f32).at[c, 0].set(1.0)

    PK = jnp.stack([jnp.kron(eyeQ, e(c, 4).T) for c in range(_C)])
    SP = jnp.stack([jnp.eye(4, dtype=f32) @ jnp.zeros((4, 16), f32)
                    .at[0, 4 * c].set(1.0).at[1, 4 * c + 1].set(1.0)
                    .at[2, 4 * c + 2].set(1.0).at[3, 4 * c + 3].set(1.0)
                    for c in range(_C)])
    UR = jnp.stack([jnp.kron(eyeQ, e(c, 4)) for c in range(_C)])
    EC = jnp.stack([jnp.zeros((8, 2), f32).at[c, 0].set(1.0)
                    .at[4 + c, 1].set(1.0) for c in range(_C)])

    consts = (
        jnp.tile(Wa, (1, _C)),
        jnp.kron(eye4, Wb),
        jnp.kron(eye4, e(0, 4) @ ones1F),
        jnp.kron(eye4, e(1, 4) @ ones1F),
        jnp.tile(b1[None, :], (1, _C)),
        jnp.tile(w1d, (1, _C)),
        jnp.kron(eye4, W2),
        jnp.kron(eye4, W2.T),
        jnp.tile(b2[None, :], (1, _C)),
        jnp.tile(Wout[:, 0][None, :], (1, _C)),
        bcmat(w1d[0]),
        bcmat(c1x), bcmat(c1y), bcmat(c2x), bcmat(c2y),
        jnp.kron(eye4, jnp.ones((_F, 1), dtype=f32) / _F),
        PK, SP, UR, EC,
    )

    def body(_, xc):
        codes = _morton_codes(xc[:, 0:2]).reshape(nb, _P)
        perm = jnp.argsort(codes, axis=1).astype(jnp.int32)
        xr = xc.reshape(nb, _P, 4)
        g = _grad_step(xr, perm[:, :, None], perm[:, None, :], consts)
        gt = g.reshape(N, 2)
        newx = xc[:, 0:2] - 0.01 * gt
        return jnp.concatenate([newx, xc[:, 2:]], axis=1)

    return jax.lax.fori_loop(0, steps, body, x)


# speedup vs baseline: 1.9005x; 1.9005x over previous
import jax
import jax.numpy as jnp
from jax.experimental import pallas as pl
from jax.experimental.pallas import tpu as pltpu

_P = 512
_R = 0.05
_TI = 32
_F = 32
_C = 4
_L = _F * _C
_Q = _P // _C


def _grad_body(xi_ref, xj4_ref, Wa4_ref, Wb16_ref, sjx_ref, sjy_ref,
               b1_4_ref, w1d4_ref, W2b_ref, W2bT_ref, b2_4_ref, wo4_ref,
               VBS_ref, V1X_ref, V1Y_ref, V2X_ref, V2Y_ref, RED4_ref,
               gi_ref, gj_ref):
    it = pl.program_id(1)
    xi = xi_ref[0]
    xj4 = xj4_ref[0]

    A4 = jnp.dot(xi, Wa4_ref[...], preferred_element_type=jnp.float32) + b1_4_ref[...]
    B4 = jnp.dot(xj4, Wb16_ref[...], preferred_element_type=jnp.float32)

    pix = jnp.broadcast_to(xi[:, 0:1], (_TI, _L))
    piy = jnp.broadcast_to(xi[:, 1:2], (_TI, _L))
    pjx = jnp.dot(xj4, sjx_ref[...], preferred_element_type=jnp.float32)
    pjy = jnp.dot(xj4, sjy_ref[...], preferred_element_type=jnp.float32)

    relx = pix[:, None, :] - pjx[None, :, :]
    rely = piy[:, None, :] - pjy[None, :, :]
    dist2 = ((pix * pix + piy * piy)[:, None, :]
             + (pjx * pjx + pjy * pjy)[None, :, :]
             - 2.0 * (pix[:, None, :] * pjx[None, :, :]
                      + piy[:, None, :] * pjy[None, :, :]))
    j_id = (4 * jax.lax.broadcasted_iota(jnp.int32, (_Q, _L), 0)
            + jax.lax.broadcasted_iota(jnp.int32, (_Q, _L), 1) // _F)
    i_id = it * _TI + jax.lax.broadcasted_iota(jnp.int32, (_TI, _Q, _L), 0)
    mask = (dist2 < _R * _R) & (i_id != j_id[None, :, :])
    r2 = relx * relx + rely * rely + 1e-8
    rdist = jax.lax.rsqrt(r2)
    dist = r2 * rdist

    z1 = A4[:, None, :] + B4[None, :, :] + dist * w1d4_ref[...][0][None, None, :]
    h = jnp.tanh(z1.astype(jnp.bfloat16)).reshape(_TI * _Q, _L)
    z2 = (jnp.dot(h, W2b_ref[...], preferred_element_type=jnp.float32)
          + b2_4_ref[...]).astype(jnp.bfloat16)
    t2 = jnp.tanh(z2)
    maskf = mask.reshape(_TI * _Q, _L)
    one = jnp.bfloat16(1.0)
    dz2 = jnp.where(maskf, (one - t2 * t2) * wo4_ref[...], jnp.bfloat16(0.0))
    dh = jnp.dot(dz2, W2bT_ref[...],
                 preferred_element_type=jnp.float32).astype(jnp.bfloat16)
    dz1 = dh * (one - h * h)

    def red(v_ref):
        r = jnp.dot(dz1, v_ref[...], preferred_element_type=jnp.float32)
        return r.reshape(_TI, _Q, _L)

    srd = red(VBS_ref) * rdist
    sux = srd * relx
    suy = srd * rely
    v1x = red(V1X_ref) + sux
    v1y = red(V1Y_ref) + suy
    v2x = red(V2X_ref) - sux
    v2y = red(V2Y_ref) - suy

    gi_x = jnp.sum(v1x, axis=(1, 2)) * (1.0 / _F)
    gi_y = jnp.sum(v1y, axis=(1, 2)) * (1.0 / _F)
    gj2x = jnp.sum(v2x, axis=0)
    gj2y = jnp.sum(v2y, axis=0)
    RED4 = RED4_ref[...]
    gj4 = jnp.concatenate(
        [jnp.dot(gj2x, RED4, preferred_element_type=jnp.float32),
         jnp.dot(gj2y, RED4, preferred_element_type=jnp.float32)], axis=1)

    gi_ref[0, 0] = jnp.stack([gi_x, gi_y], axis=0)

    @pl.when(it == 0)
    def _():
        gj_ref[...] = jnp.zeros_like(gj_ref)

    gj_ref[0] = gj_ref[0] + gj4


def _grad_step(xr, xr4, consts):
    nb = xr.shape[0]
    grid = (nb, _P // _TI)

    def wspec(a):
        return pl.BlockSpec(a.shape, lambda b, it: (0,) * a.ndim)

    gi, gj = pl.pallas_call(
        _grad_body,
        grid=grid,
        in_specs=[
            pl.BlockSpec((1, _TI, 4), lambda b, it: (b, it, 0)),
            pl.BlockSpec((1, _Q, 16), lambda b, it: (b, 0, 0)),
        ] + [wspec(c) for c in consts],
        out_specs=[
            pl.BlockSpec((1, 1, 2, _TI), lambda b, it: (b, it, 0, 0)),
            pl.BlockSpec((1, _Q, 8), lambda b, it: (b, 0, 0)),
        ],
        out_shape=[
            jax.ShapeDtypeStruct((nb, _P // _TI, 2, _TI), jnp.float32),
            jax.ShapeDtypeStruct((nb, _Q, 8), jnp.float32),
        ],
        compiler_params=pltpu.CompilerParams(
            dimension_semantics=("parallel", "arbitrary")),
    )(xr, xr4, *consts)
    return gi, gj


def kernel(x, batch, steps, W1, b1, W2, b2, Wout, bout):
    N = x.shape[0]
    nb = N // _P
    f32 = jnp.float32

    Wr = W1[8:10]
    pad = jnp.zeros((2, _F), dtype=f32)
    Wa = W1[0:4] + jnp.concatenate([Wr, pad], axis=0)
    Wb = W1[4:8] - jnp.concatenate([Wr, pad], axis=0)
    w1d = W1[10:11]
    c1x = W1[0] + W1[8]
    c1y = W1[1] + W1[9]
    c2x = W1[4] - W1[8]
    c2y = W1[5] - W1[9]

    eye4 = jnp.eye(_C, dtype=f32)
    ones1F = jnp.ones((1, _F), dtype=f32)

    def bcmat(vec):
        return jnp.kron(eye4, vec[:, None] @ ones1F)

    e0 = jnp.zeros((4, 1), dtype=f32).at[0, 0].set(1.0)
    e1 = jnp.zeros((4, 1), dtype=f32).at[1, 0].set(1.0)

    consts = (
        jnp.tile(Wa, (1, _C)),
        jnp.kron(eye4, Wb),
        jnp.kron(eye4, e0 @ ones1F),
        jnp.kron(eye4, e1 @ ones1F),
        jnp.tile(b1[None, :], (1, _C)),
        jnp.tile(w1d, (1, _C)),
        jnp.kron(eye4, W2).astype(jnp.bfloat16),
        jnp.kron(eye4, W2.T).astype(jnp.bfloat16),
        jnp.tile(b2[None, :], (1, _C)),
        jnp.tile(Wout[:, 0][None, :], (1, _C)).astype(jnp.bfloat16),
        bcmat(w1d[0]).astype(jnp.bfloat16),
        bcmat(c1x).astype(jnp.bfloat16), bcmat(c1y).astype(jnp.bfloat16),
        bcmat(c2x).astype(jnp.bfloat16), bcmat(c2y).astype(jnp.bfloat16),
        jnp.kron(eye4, jnp.ones((_F, 1), dtype=f32) / _F),
    )

    def body(_, xc):
        xr = xc.reshape(nb, _P, 4)
        xr4 = xc.reshape(nb, _Q, 16)
        gi, gj = _grad_step(xr, xr4, consts)
        gix = jnp.transpose(gi, (0, 2, 1, 3)).reshape(nb, 2, _P)
        gjx = gj[:, :, 0:4].reshape(nb, _P)
        gjy = gj[:, :, 4:8].reshape(nb, _P)
        gx = (gix[:, 0] + gjx).reshape(N)
        gy = (gix[:, 1] + gjy).reshape(N)
        newx = xc[:, 0:2] - 0.01 * jnp.stack([gx, gy], axis=1)
        return jnp.concatenate([newx, xc[:, 2:]], axis=1)

    return jax.lax.fori_loop(0, steps, body, x)
